# Initial kernel scaffold; baseline (speedup 1.0000x reference)
import jax
import jax.numpy as jnp
from jax.experimental import pallas as pl


def kernel(node_embedding, edge_probs, send_edges, recv_edges, node_masks, W1, b1, W2, b2):
    raise NotImplementedError("write your pallas kernel here")


# speedup vs baseline: 2.8131x; 2.8131x over previous
import functools

import jax
import jax.numpy as jnp
from jax import lax
from jax.experimental import pallas as pl
from jax.experimental.pallas import tpu as pltpu
from jax.experimental.pallas import tpu_sc as plsc

A, E, H = 10000, 320000, 128
NC, NS = 2, 16
NW = NC * NS
EPW = E // NW
CH = 80
NCHUNK = EPW // CH
RPS = 624
TAIL0 = NS * RPS
TAILN = A - TAIL0

@functools.cache
def _sc_kernels():
    mesh = plsc.VectorSubcoreMesh(core_axis_name="c", subcore_axis_name="s")

    @functools.partial(
        pl.kernel,
        out_type=(jax.ShapeDtypeStruct((E, H), jnp.float32),
                  jax.ShapeDtypeStruct((E, H), jnp.float32)),
        mesh=mesh,
        scratch_types=[
            pltpu.VMEM((CH,), jnp.int32),
            pltpu.VMEM((CH,), jnp.int32),
            pltpu.VMEM((CH, H), jnp.float32),
            pltpu.VMEM((CH, H), jnp.float32),
            pltpu.SemaphoreType.DMA,
        ],
    )
    def _sc_gather(node_hbm, send_hbm, recv_hbm, sout_hbm, rout_hbm,
                   sidx_v, ridx_v, srow_v, rrow_v, sem):
        c = lax.axis_index("c")
        s = lax.axis_index("s")
        base = (s * NC + c) * EPW

        @pl.loop(0, NCHUNK)
        def _(i):
            off = base + i * CH
            pltpu.sync_copy(send_hbm.at[pl.ds(off, CH)], sidx_v)
            pltpu.sync_copy(recv_hbm.at[pl.ds(off, CH)], ridx_v)
            c1 = pltpu.async_copy(node_hbm.at[sidx_v], srow_v, sem)
            c2 = pltpu.async_copy(node_hbm.at[ridx_v], rrow_v, sem)
            c1.wait()
            c2.wait()
            pltpu.sync_copy(srow_v, sout_hbm.at[pl.ds(off, CH)])
            pltpu.sync_copy(rrow_v, rout_hbm.at[pl.ds(off, CH)])

    @functools.partial(
        pl.kernel,
        out_type=jax.ShapeDtypeStruct((NC, A, H), jnp.float32),
        mesh=mesh,
        scratch_types=[
            pltpu.VMEM((1, CH), jnp.int32),
            pltpu.VMEM((CH, H), jnp.float32),
            pltpu.VMEM_SHARED((A, H), jnp.float32),
            pltpu.SemaphoreType.DMA,
        ],
    )
    def _sc_scatter(msg_hbm, recv_hbm, zero_hbm, out_hbm,
                    idx_v, row_v, acc_sh, sem):
        c = lax.axis_index("c")
        s = lax.axis_index("s")
        base = (s * NC + c) * EPW
        r0 = s * RPS
        pltpu.sync_copy(zero_hbm.at[pl.ds(r0, RPS)], acc_sh.at[pl.ds(r0, RPS)])

        @pl.when(s == 0)
        def _():
            pltpu.sync_copy(zero_hbm.at[pl.ds(TAIL0, TAILN)],
                            acc_sh.at[pl.ds(TAIL0, TAILN)])

        plsc.subcore_barrier()

        @pl.loop(0, NCHUNK)
        def _(i):
            off = base + i * CH
            pltpu.sync_copy(recv_hbm.at[pl.ds(off, CH)], idx_v.at[0])
            pltpu.sync_copy(msg_hbm.at[pl.ds(off, CH)], row_v)
            pltpu.sync_copy(row_v, acc_sh.at[idx_v.at[0]], add=True)

        plsc.subcore_barrier()
        pltpu.sync_copy(acc_sh.at[pl.ds(r0, RPS)], out_hbm.at[c, pl.ds(r0, RPS)])

        @pl.when(s == 0)
        def _():
            pltpu.sync_copy(acc_sh.at[pl.ds(TAIL0, TAILN)],
                            out_hbm.at[c, pl.ds(TAIL0, TAILN)])

    return _sc_gather, _sc_scatter


BE = 2000


def _mlp_body(se, re, p, w1a, w1b, b1, w2, b2, o):
    h = jnp.tanh(
        jnp.dot(se[...], w1a[...], preferred_element_type=jnp.float32)
        + jnp.dot(re[...], w1b[...], preferred_element_type=jnp.float32)
        + b1[...])
    m = jnp.tanh(jnp.dot(h, w2[...], preferred_element_type=jnp.float32) + b2[...])
    o[...] = m * p[...]


def _tc_mlp(send_emb, recv_emb, p, w1a, w1b, b1, w2, b2):
    return pl.pallas_call(
        _mlp_body,
        grid=(E // BE,),
        in_specs=[
            pl.BlockSpec((BE, H), lambda i: (i, 0)),
            pl.BlockSpec((BE, H), lambda i: (i, 0)),
            pl.BlockSpec((BE, 1), lambda i: (i, 0)),
            pl.BlockSpec((H, H), lambda i: (0, 0)),
            pl.BlockSpec((H, H), lambda i: (0, 0)),
            pl.BlockSpec((1, H), lambda i: (0, 0)),
            pl.BlockSpec((H, H), lambda i: (0, 0)),
            pl.BlockSpec((1, H), lambda i: (0, 0)),
        ],
        out_specs=pl.BlockSpec((BE, H), lambda i: (i, 0)),
        out_shape=jax.ShapeDtypeStruct((E, H), jnp.float32),
    )(send_emb, recv_emb, p, w1a, w1b, b1, w2, b2)


def _add_body(a, o):
    o[...] = a[0] + a[1]


def _tc_add(partials):
    return pl.pallas_call(
        _add_body,
        grid=(10,),
        in_specs=[pl.BlockSpec((NC, A // 10, H), lambda i: (0, i, 0))],
        out_specs=pl.BlockSpec((A // 10, H), lambda i: (i, 0)),
        out_shape=jax.ShapeDtypeStruct((A, H), jnp.float32),
    )(partials)


def kernel(node_embedding, edge_probs, send_edges, recv_edges, node_masks,
           W1, b1, W2, b2):
    del node_masks
    x = node_embedding[0]
    p = edge_probs[0, :, 1:2]
    _sc_gather, _sc_scatter = _sc_kernels()
    send_emb, recv_emb = _sc_gather(x, send_edges, recv_edges)
    msg = _tc_mlp(send_emb, recv_emb, p,
                  W1[:H], W1[H:], b1.reshape(1, H), W2, b2.reshape(1, H))
    zeros = jnp.zeros((A, H), jnp.float32)
    partials = _sc_scatter(msg, recv_edges, zeros)
    return _tc_add(partials)[None]


# speedup vs baseline: 4.1266x; 1.4669x over previous
import functools

import jax
import jax.numpy as jnp
from jax import lax
from jax.experimental import pallas as pl
from jax.experimental.pallas import tpu as pltpu
from jax.experimental.pallas import tpu_sc as plsc

A, E, H = 10000, 320000, 128
NC, NS = 2, 16
NW = NC * NS
EPW = E // NW
CH = 128
NCH = 78
ETAIL = EPW - NCH * CH
RPS = 624
TAIL0 = NS * RPS
TAILN = A - TAIL0

@functools.cache
def _sc_kernels():
    mesh = plsc.VectorSubcoreMesh(core_axis_name="c", subcore_axis_name="s")

    @functools.partial(
        pl.kernel,
        out_type=(jax.ShapeDtypeStruct((E, H), jnp.float32),
                  jax.ShapeDtypeStruct((E, H), jnp.float32)),
        mesh=mesh,
        scratch_types=[
            pltpu.VMEM((2, CH), jnp.int32),
            pltpu.VMEM((2, CH), jnp.int32),
            pltpu.VMEM((CH, H), jnp.float32),
            pltpu.VMEM((CH, H), jnp.float32),
            pltpu.VMEM((CH, H), jnp.float32),
            pltpu.VMEM((CH, H), jnp.float32),
            pltpu.SemaphoreType.DMA,
            pltpu.SemaphoreType.DMA,
            pltpu.SemaphoreType.DMA,
            pltpu.SemaphoreType.DMA,
        ],
    )
    def _sc_gather(node_hbm, send_hbm, recv_hbm, sout_hbm, rout_hbm,
                   sidx_v, ridx_v, sr0, sr1, rr0, rr1,
                   sg0, sg1, sw0, sw1):
        c = lax.axis_index("c")
        s = lax.axis_index("s")
        base = (s * NC + c) * EPW
        srow = (sr0, sr1)
        rrow = (rr0, rr1)
        sgs = (sg0, sg1)
        sws = (sw0, sw1)

        def load_idx(chunk, b):
            off = base + chunk * CH
            pltpu.sync_copy(send_hbm.at[pl.ds(off, CH)], sidx_v.at[b])
            pltpu.sync_copy(recv_hbm.at[pl.ds(off, CH)], ridx_v.at[b])

        def fire_gather(b):
            pltpu.async_copy(node_hbm.at[sidx_v.at[b]], srow[b], sgs[b])
            pltpu.async_copy(node_hbm.at[ridx_v.at[b]], rrow[b], sgs[b])

        def wait_gather(b):
            pltpu.make_async_copy(node_hbm.at[sidx_v.at[b]], srow[b], sgs[b]).wait()
            pltpu.make_async_copy(node_hbm.at[ridx_v.at[b]], rrow[b], sgs[b]).wait()

        load_idx(0, 0)
        fire_gather(0)
        load_idx(1, 1)
        fire_gather(1)

        @pl.loop(0, (NCH - 2) // 2)
        def _(j):
            for b in (0, 1):
                i = 2 * j + b
                off = base + i * CH
                wait_gather(b)
                w1 = pltpu.async_copy(srow[b], sout_hbm.at[pl.ds(off, CH)], sws[b])
                w2 = pltpu.async_copy(rrow[b], rout_hbm.at[pl.ds(off, CH)], sws[b])
                load_idx(i + 2, b)
                w1.wait()
                w2.wait()
                fire_gather(b)

        for b in (0, 1):
            i = NCH - 2 + b
            off = base + i * CH
            wait_gather(b)
            pltpu.sync_copy(srow[b], sout_hbm.at[pl.ds(off, CH)])
            pltpu.sync_copy(rrow[b], rout_hbm.at[pl.ds(off, CH)])

        toff = base + NCH * CH
        pltpu.sync_copy(send_hbm.at[pl.ds(toff, ETAIL)],
                        sidx_v.at[0, pl.ds(0, ETAIL)])
        pltpu.sync_copy(recv_hbm.at[pl.ds(toff, ETAIL)],
                        ridx_v.at[0, pl.ds(0, ETAIL)])
        t1 = pltpu.async_copy(node_hbm.at[sidx_v.at[0, pl.ds(0, ETAIL)]],
                              sr0.at[pl.ds(0, ETAIL)], sg0)
        t2 = pltpu.async_copy(node_hbm.at[ridx_v.at[0, pl.ds(0, ETAIL)]],
                              rr0.at[pl.ds(0, ETAIL)], sg0)
        t1.wait()
        t2.wait()
        pltpu.sync_copy(sr0.at[pl.ds(0, ETAIL)], sout_hbm.at[pl.ds(toff, ETAIL)])
        pltpu.sync_copy(rr0.at[pl.ds(0, ETAIL)], rout_hbm.at[pl.ds(toff, ETAIL)])

    @functools.partial(
        pl.kernel,
        out_type=jax.ShapeDtypeStruct((NC, A, H), jnp.float32),
        mesh=mesh,
        scratch_types=[
            pltpu.VMEM((2, CH), jnp.int32),
            pltpu.VMEM((1, ETAIL), jnp.int32),
            pltpu.VMEM((CH, H), jnp.float32),
            pltpu.VMEM((CH, H), jnp.float32),
            pltpu.VMEM_SHARED((A, H), jnp.float32),
            pltpu.SemaphoreType.DMA,
            pltpu.SemaphoreType.DMA,
            pltpu.SemaphoreType.DMA,
            pltpu.SemaphoreType.DMA,
        ],
    )
    def _sc_scatter(msg_hbm, recv_hbm, zero_hbm, out_hbm,
                    idx_v, idx_t, mr0, mr1, acc_sh, sl0, sl1, sa0, sa1):
        c = lax.axis_index("c")
        s = lax.axis_index("s")
        base = (s * NC + c) * EPW
        r0 = s * RPS
        mrow = (mr0, mr1)
        sls = (sl0, sl1)
        sas = (sa0, sa1)
        pltpu.sync_copy(zero_hbm.at[pl.ds(r0, RPS)], acc_sh.at[pl.ds(r0, RPS)])

        @pl.when(s == 0)
        def _():
            pltpu.sync_copy(zero_hbm.at[pl.ds(TAIL0, TAILN)],
                            acc_sh.at[pl.ds(TAIL0, TAILN)])

        plsc.subcore_barrier()

        def fire_load(chunk, b):
            off = base + chunk * CH
            pltpu.async_copy(recv_hbm.at[pl.ds(off, CH)], idx_v.at[b], sls[b])
            pltpu.async_copy(msg_hbm.at[pl.ds(off, CH)], mrow[b], sls[b])

        def wait_load(chunk, b):
            off = base + chunk * CH
            pltpu.make_async_copy(recv_hbm.at[pl.ds(off, CH)], idx_v.at[b],
                                  sls[b]).wait()
            pltpu.make_async_copy(msg_hbm.at[pl.ds(off, CH)], mrow[b],
                                  sls[b]).wait()

        fire_load(0, 0)
        fire_load(1, 1)

        @pl.loop(0, (NCH - 2) // 2)
        def _(j):
            for b in (0, 1):
                i = 2 * j + b
                wait_load(i, b)
                a = pltpu.async_copy(mrow[b], acc_sh.at[idx_v.at[b]],
                                     sas[b], add=True)
                a.wait()
                fire_load(i + 2, b)

        for b in (0, 1):
            wait_load(NCH - 2 + b, b)
            pltpu.sync_copy(mrow[b], acc_sh.at[idx_v.at[b]], add=True)

        toff = base + NCH * CH
        pltpu.sync_copy(recv_hbm.at[pl.ds(toff, ETAIL)], idx_t.at[0])
        pltpu.sync_copy(msg_hbm.at[pl.ds(toff, ETAIL)], mr0.at[pl.ds(0, ETAIL)])
        pltpu.sync_copy(mr0.at[pl.ds(0, ETAIL)],
                        acc_sh.at[idx_t.at[0]], add=True)

        plsc.subcore_barrier()
        pltpu.sync_copy(acc_sh.at[pl.ds(r0, RPS)], out_hbm.at[c, pl.ds(r0, RPS)])

        @pl.when(s == 0)
        def _():
            pltpu.sync_copy(acc_sh.at[pl.ds(TAIL0, TAILN)],
                            out_hbm.at[c, pl.ds(TAIL0, TAILN)])

    return _sc_gather, _sc_scatter


BE = 2000


def _mlp_body(se, re, p, w1a, w1b, b1, w2, b2, o):
    h = jnp.tanh(
        jnp.dot(se[...], w1a[...], preferred_element_type=jnp.float32)
        + jnp.dot(re[...], w1b[...], preferred_element_type=jnp.float32)
        + b1[...])
    m = jnp.tanh(jnp.dot(h, w2[...], preferred_element_type=jnp.float32) + b2[...])
    o[...] = m * p[...]


def _tc_mlp(send_emb, recv_emb, p, w1a, w1b, b1, w2, b2):
    return pl.pallas_call(
        _mlp_body,
        grid=(E // BE,),
        in_specs=[
            pl.BlockSpec((BE, H), lambda i: (i, 0)),
            pl.BlockSpec((BE, H), lambda i: (i, 0)),
            pl.BlockSpec((BE, 1), lambda i: (i, 0)),
            pl.BlockSpec((H, H), lambda i: (0, 0)),
            pl.BlockSpec((H, H), lambda i: (0, 0)),
            pl.BlockSpec((1, H), lambda i: (0, 0)),
            pl.BlockSpec((H, H), lambda i: (0, 0)),
            pl.BlockSpec((1, H), lambda i: (0, 0)),
        ],
        out_specs=pl.BlockSpec((BE, H), lambda i: (i, 0)),
        out_shape=jax.ShapeDtypeStruct((E, H), jnp.float32),
    )(send_emb, recv_emb, p, w1a, w1b, b1, w2, b2)


def _add_body(a, o):
    o[...] = a[0] + a[1]


def _tc_add(partials):
    return pl.pallas_call(
        _add_body,
        grid=(10,),
        in_specs=[pl.BlockSpec((NC, A // 10, H), lambda i: (0, i, 0))],
        out_specs=pl.BlockSpec((A // 10, H), lambda i: (i, 0)),
        out_shape=jax.ShapeDtypeStruct((A, H), jnp.float32),
    )(partials)


def kernel(node_embedding, edge_probs, send_edges, recv_edges, node_masks,
           W1, b1, W2, b2):
    del node_masks
    x = node_embedding[0]
    p = edge_probs[0, :, 1:2]
    _sc_gather, _sc_scatter = _sc_kernels()
    send_emb, recv_emb = _sc_gather(x, send_edges, recv_edges)
    msg = _tc_mlp(send_emb, recv_emb, p,
                  W1[:H], W1[H:], b1.reshape(1, H), W2, b2.reshape(1, H))
    zeros = jnp.zeros((A, H), jnp.float32)
    partials = _sc_scatter(msg, recv_edges, zeros)
    return _tc_add(partials)[None]
